# baseline (speedup 1.0000x reference)
import functools
import math

import jax
import jax.numpy as jnp
from jax import lax
from jax.experimental import pallas as pl
from jax.experimental.pallas import tpu as pltpu
from jax.experimental.pallas import tpu_sc as plsc

D_MODEL = 1024
B_TOTAL = 4 * 4096
NUM_CORES = 2
NUM_SUBCORES = 16
NW = NUM_CORES * NUM_SUBCORES
BPW = B_TOTAL // NW
CHUNK = 16
NCHUNK = BPW // CHUNK
NBUF = 6
LOOKAHEAD = 2
LANES = 16
SCALE = math.sqrt(D_MODEL)

_mesh = plsc.VectorSubcoreMesh(core_axis_name="c", subcore_axis_name="s")


@functools.partial(
    pl.kernel,
    mesh=_mesh,
    out_type=jax.ShapeDtypeStruct((B_TOTAL, D_MODEL), jnp.float32),
    scratch_types=[
        pltpu.VMEM((BPW,), jnp.int32),
        pltpu.VMEM((NBUF, CHUNK, D_MODEL), jnp.float32),
        pltpu.SemaphoreType.DMA((NBUF,)),
        pltpu.SemaphoreType.DMA((NBUF,)),
    ],
)
def _emb_lookup(idx_hbm, table_hbm, out_hbm, idx_v, rows_v, gsems, ssems):
    wid = lax.axis_index("s") * NUM_CORES + lax.axis_index("c")
    base = wid * BPW
    pltpu.sync_copy(idx_hbm.at[pl.ds(base, BPW)], idx_v)

    def gather(g, b):
        return pltpu.async_copy(
            table_hbm.at[idx_v.at[pl.ds(g * CHUNK, CHUNK)]],
            rows_v.at[b],
            gsems.at[b],
        )

    def scale(b):
        def row_body(r, c2):
            for col in range(0, D_MODEL, LANES):
                rows_v[b, r, pl.ds(col, LANES)] = (
                    rows_v[b, r, pl.ds(col, LANES)] * SCALE
                )
            return c2

        lax.fori_loop(0, CHUNK, row_body, 0)

    pending = [None] * NBUF
    inflight = [None] * NBUF
    for g in range(min(LOOKAHEAD, NCHUNK)):
        inflight[g % NBUF] = gather(g, g % NBUF)
    for g in range(NCHUNK):
        b = g % NBUF
        nxt = g + LOOKAHEAD
        if nxt < NCHUNK:
            nb = nxt % NBUF
            if pending[nb] is not None:
                pending[nb].wait()
            inflight[nb] = gather(nxt, nb)
        inflight[b].wait()
        scale(b)
        pending[b] = pltpu.async_copy(
            rows_v.at[b], out_hbm.at[pl.ds(base + g * CHUNK, CHUNK)], ssems.at[b]
        )
    for p in pending:
        if p is not None:
            p.wait()


def kernel(x, table):
    idx = jnp.reshape(x, (B_TOTAL,)).astype(jnp.int32)
    out = _emb_lookup(idx, table)
    return jnp.reshape(out, (*x.shape, D_MODEL))
